# Initial kernel scaffold; baseline (speedup 1.0000x reference)
import jax
import jax.numpy as jnp
from jax.experimental import pallas as pl


def kernel(token_ids, weight):
    raise NotImplementedError("write your pallas kernel here")


# speedup vs baseline: 2.9704x; 2.9704x over previous
import jax
import jax.numpy as jnp
from jax import lax
from jax.experimental import pallas as pl
from jax.experimental.pallas import tpu as pltpu
from jax.experimental.pallas import tpu_sc as plsc

NUM_CORES = 2
NUM_SUBCORES = 16
NUM_WORKERS = NUM_CORES * NUM_SUBCORES
CHUNK = 128


def _gather_body(idx_hbm, table_hbm, out_hbm, idx_v, rows_v, sem):
    wid = lax.axis_index("s") * NUM_CORES + lax.axis_index("c")
    rows_per_worker = idx_hbm.shape[1]
    base = wid * rows_per_worker
    pltpu.sync_copy(idx_hbm.at[wid], idx_v)

    def chunk(j, carry):
        pltpu.async_copy(table_hbm.at[idx_v.at[j]], rows_v, sem).wait()
        pltpu.sync_copy(rows_v, out_hbm.at[pl.ds((base + j) * CHUNK, CHUNK)])
        return carry

    lax.fori_loop(0, rows_per_worker, chunk, 0)


def kernel(token_ids, weight):
    B, S = token_ids.shape
    V, D = weight.shape
    n = B * S
    rows_per_worker = n // CHUNK // NUM_WORKERS
    idx3d = token_ids.astype(jnp.int32).reshape(NUM_WORKERS, rows_per_worker, CHUNK)
    mesh = plsc.VectorSubcoreMesh(core_axis_name="c", subcore_axis_name="s")
    out = pl.kernel(
        _gather_body,
        out_type=jax.ShapeDtypeStruct((n, D), weight.dtype),
        mesh=mesh,
        scratch_types=[
            pltpu.VMEM((rows_per_worker, CHUNK), jnp.int32),
            pltpu.VMEM((CHUNK, D), jnp.float32),
            pltpu.SemaphoreType.DMA,
        ],
    )(idx3d, weight)
    return out.reshape(B, S, D)


# speedup vs baseline: 3.3506x; 1.1280x over previous
import jax
import jax.numpy as jnp
from jax import lax
from jax.experimental import pallas as pl
from jax.experimental.pallas import tpu as pltpu
from jax.experimental.pallas import tpu_sc as plsc

NUM_CORES = 2
NUM_SUBCORES = 16
NUM_WORKERS = NUM_CORES * NUM_SUBCORES
CHUNK = 128
NBUF = 5
LOOKAHEAD = 3


def _gather_body(idx_hbm, table_hbm, out_hbm, idx_v,
                 b0, b1, b2, b3, b4,
                 g0, g1, g2, g3, g4,
                 w0, w1, w2, w3, w4):
    bufs = [b0, b1, b2, b3, b4]
    gsems = [g0, g1, g2, g3, g4]
    wsems = [w0, w1, w2, w3, w4]

    wid = lax.axis_index("s") * NUM_CORES + lax.axis_index("c")
    rpw = idx_hbm.shape[1]
    base = wid * rpw
    pltpu.sync_copy(idx_hbm.at[wid], idx_v)

    def gather_start(j, b):
        pltpu.async_copy(table_hbm.at[idx_v.at[j]], bufs[b], gsems[b])

    def gather_wait(b):
        pltpu.make_async_copy(table_hbm.at[pl.ds(0, CHUNK)], bufs[b],
                              gsems[b]).wait()

    def writeback_wait(b):
        pltpu.make_async_copy(table_hbm.at[pl.ds(0, CHUNK)], bufs[b],
                              wsems[b]).wait()

    for b in range(LOOKAHEAD):
        gather_start(b, b)

    nrounds = rpw // NBUF

    def round_body(r, carry):
        for b in range(NBUF):
            j = r * NBUF + b
            gather_wait(b)
            pltpu.async_copy(
                bufs[b], out_hbm.at[pl.ds((base + j) * CHUNK, CHUNK)],
                wsems[b])
            bn = (b + LOOKAHEAD) % NBUF
            jn = j + LOOKAHEAD

            @pl.when(jnp.logical_and(jn >= NBUF, jn < rpw))
            def _():
                writeback_wait(bn)
                gather_start(jn, bn)

            @pl.when(jn < NBUF)
            def _():
                gather_start(jn, bn)
        return carry

    lax.fori_loop(0, nrounds, round_body, 0)

    for b in range(NBUF):
        writeback_wait(b)


def kernel(token_ids, weight):
    B, S = token_ids.shape
    V, D = weight.shape
    n = B * S
    rows_per_worker = n // CHUNK // NUM_WORKERS
    idx3d = token_ids.astype(jnp.int32).reshape(NUM_WORKERS, rows_per_worker, CHUNK)
    mesh = plsc.VectorSubcoreMesh(core_axis_name="c", subcore_axis_name="s")
    out = pl.kernel(
        _gather_body,
        out_type=jax.ShapeDtypeStruct((n, D), weight.dtype),
        mesh=mesh,
        scratch_types=(
            [pltpu.VMEM((rows_per_worker, CHUNK), jnp.int32)]
            + [pltpu.VMEM((CHUNK, D), jnp.float32) for _ in range(NBUF)]
            + [pltpu.SemaphoreType.DMA for _ in range(2 * NBUF)]
        ),
    )(idx3d, weight)
    return out.reshape(B, S, D)


# speedup vs baseline: 5.9763x; 1.7836x over previous
import jax
import jax.numpy as jnp
from jax import lax
from jax.experimental import pallas as pl
from jax.experimental.pallas import tpu as pltpu
from jax.experimental.pallas import tpu_sc as plsc

NUM_CORES = 2
NUM_SUBCORES = 16
NUM_WORKERS = NUM_CORES * NUM_SUBCORES
NBUF = 8
LOOKAHEAD = 7


def _gather_body(idx_hbm, table_hbm, out_hbm, idx_v, bufs, sems):
    S = idx_hbm.shape[1]
    rpw = idx_hbm.shape[0] // NUM_WORKERS
    wid = lax.axis_index("s") * NUM_CORES + lax.axis_index("c")
    base = wid * rpw
    pltpu.sync_copy(idx_hbm.at[pl.ds(base, rpw)], idx_v)

    gsems = sems

    def gather_start(r, b):
        pltpu.async_copy(table_hbm.at[idx_v.at[r]], bufs[b], gsems[b])

    def gather_wait(b):
        pltpu.make_async_copy(out_hbm.at[0], bufs[b], gsems[b]).wait()

    for b in range(LOOKAHEAD):
        gather_start(b, b)

    nrounds = rpw // NBUF

    def round_body(q, carry):
        for b in range(NBUF):
            r = q * NBUF + b
            gather_wait(b)
            pltpu.sync_copy(bufs[b], out_hbm.at[base + r])
            bn = (b + LOOKAHEAD) % NBUF
            rn = r + LOOKAHEAD

            @pl.when(rn < rpw)
            def _():
                gather_start(rn, bn)
        return carry

    lax.fori_loop(0, nrounds, round_body, 0)


def kernel(token_ids, weight):
    B, S = token_ids.shape
    V, D = weight.shape
    rows_per_worker = B // NUM_WORKERS
    idx = token_ids.astype(jnp.int32)
    mesh = plsc.VectorSubcoreMesh(core_axis_name="c", subcore_axis_name="s")

    def body(idx_hbm, table_hbm, out_hbm, idx_v, *rest):
        _gather_body(idx_hbm, table_hbm, out_hbm, idx_v,
                     list(rest[:NBUF]), list(rest[NBUF:]))

    out = pl.kernel(
        body,
        out_type=jax.ShapeDtypeStruct((B, S, D), weight.dtype),
        mesh=mesh,
        scratch_types=(
            [pltpu.VMEM((rows_per_worker, S), jnp.int32)]
            + [pltpu.VMEM((S, D), jnp.float32) for _ in range(NBUF)]
            + [pltpu.SemaphoreType.DMA for _ in range(NBUF)]
        ),
    )(idx, weight)
    return out


# speedup vs baseline: 5.9840x; 1.0013x over previous
import jax
import jax.numpy as jnp
from jax import lax
from jax.experimental import pallas as pl
from jax.experimental.pallas import tpu as pltpu
from jax.experimental.pallas import tpu_sc as plsc

NUM_CORES = 2
NUM_SUBCORES = 16
NUM_WORKERS = NUM_CORES * NUM_SUBCORES
NBUF = 8
LOOKAHEAD = 7


def _gather_body(idx_hbm, table_hbm, out_hbm, idx_v, bufs, sems):
    S = idx_hbm.shape[1]
    rpw = idx_hbm.shape[0] // NUM_WORKERS
    wid = lax.axis_index("s") * NUM_CORES + lax.axis_index("c")
    base = wid * rpw
    pltpu.sync_copy(idx_hbm.at[pl.ds(base, rpw)], idx_v)

    gsems = sems

    def gather_start(r, b):
        pltpu.async_copy(table_hbm.at[idx_v.at[r]], bufs[b], gsems[b])

    def gather_wait(b):
        pltpu.make_async_copy(out_hbm.at[0], bufs[b], gsems[b]).wait()

    for b in range(LOOKAHEAD):
        gather_start(b, b)

    nrounds = rpw // NBUF

    def round_body(q, carry):
        for b in range(NBUF):
            r = q * NBUF + b
            gather_wait(b)
            pltpu.sync_copy(bufs[b], out_hbm.at[base + r])
            bn = (b + LOOKAHEAD) % NBUF
            rn = r + LOOKAHEAD

            @pl.when(rn < rpw)
            def _():
                gather_start(rn, bn)
        return carry

    lax.fori_loop(0, nrounds, round_body, 0)


def kernel(token_ids, weight):
    B, S = token_ids.shape
    V, D = weight.shape
    rows_per_worker = B // NUM_WORKERS
    idx = token_ids.astype(jnp.int32)
    mesh = plsc.VectorSubcoreMesh(core_axis_name="c", subcore_axis_name="s")

    def body(idx_hbm, table_hbm, out_hbm, idx_v, *rest):
        _gather_body(idx_hbm, table_hbm, out_hbm, idx_v,
                     list(rest[:NBUF]), list(rest[NBUF:]))

    out = pl.kernel(
        body,
        out_type=jax.ShapeDtypeStruct((B, S, D), weight.dtype),
        mesh=mesh,
        compiler_params=pltpu.CompilerParams(use_tc_tiling_on_sc=True),
        scratch_types=(
            [pltpu.VMEM((rows_per_worker, S), jnp.int32)]
            + [pltpu.VMEM((S, D), jnp.float32) for _ in range(NBUF)]
            + [pltpu.SemaphoreType.DMA for _ in range(NBUF)]
        ),
    )(idx, weight)
    return out


# speedup vs baseline: 10.4373x; 1.7442x over previous
import jax
import jax.numpy as jnp
from jax import lax
from jax.experimental import pallas as pl
from jax.experimental.pallas import tpu as pltpu
from jax.experimental.pallas import tpu_sc as plsc

NUM_CORES = 2
NUM_SUBCORES = 16
NUM_WORKERS = NUM_CORES * NUM_SUBCORES
CHUNK = 128
NBUF = 5
LOOKAHEAD = 4


def _gather_body(idx_hbm, table_hbm, out_hbm, idx_v, bufs, gsems):
    wid = lax.axis_index("s") * NUM_CORES + lax.axis_index("c")
    rpw = idx_hbm.shape[1]
    base = wid * rpw
    pltpu.sync_copy(idx_hbm.at[wid], idx_v)

    def gather_start(j, b):
        pltpu.async_copy(table_hbm.at[idx_v.at[j]], bufs[b], gsems[b])

    def gather_wait(b):
        pltpu.make_async_copy(table_hbm.at[pl.ds(0, CHUNK)], bufs[b],
                              gsems[b]).wait()

    for b in range(LOOKAHEAD):
        gather_start(b, b)

    nrounds = rpw // NBUF

    def round_body(r, carry):
        for b in range(NBUF):
            j = r * NBUF + b
            gather_wait(b)
            pltpu.sync_copy(bufs[b], out_hbm.at[pl.ds((base + j) * CHUNK, CHUNK)])
            bn = (b + LOOKAHEAD) % NBUF
            jn = j + LOOKAHEAD

            @pl.when(jn < rpw)
            def _():
                gather_start(jn, bn)
        return carry

    lax.fori_loop(0, nrounds, round_body, 0)


def kernel(token_ids, weight):
    B, S = token_ids.shape
    V, D = weight.shape
    n = B * S
    rows_per_worker = n // CHUNK // NUM_WORKERS
    idx3d = token_ids.astype(jnp.int32).T.reshape(
        NUM_WORKERS, rows_per_worker, CHUNK)
    mesh = plsc.VectorSubcoreMesh(core_axis_name="c", subcore_axis_name="s")

    def body(idx_hbm, table_hbm, out_hbm, idx_v, *rest):
        _gather_body(idx_hbm, table_hbm, out_hbm, idx_v,
                     list(rest[:NBUF]), list(rest[NBUF:]))

    out = pl.kernel(
        body,
        out_type=jax.ShapeDtypeStruct((n, D), weight.dtype),
        mesh=mesh,
        scratch_types=(
            [pltpu.VMEM((rows_per_worker, CHUNK), jnp.int32)]
            + [pltpu.VMEM((CHUNK, D), jnp.float32) for _ in range(NBUF)]
            + [pltpu.SemaphoreType.DMA for _ in range(NBUF)]
        ),
    )(idx3d, weight)
    return out.reshape(S, B, D).transpose(1, 0, 2)
